# Initial kernel scaffold; baseline (speedup 1.0000x reference)
import functools

import jax
import jax.numpy as jnp
from jax import lax
from jax.experimental import pallas as pl
from jax.experimental.pallas import tpu as pltpu
from jax.experimental.pallas import tpu_sc as plsc

N_NODES = 10000
N_EDGES = 320000
D_IN = 128
D_HID = 128
D_OUT = 40

NC, NS = 2, 16
NW = NC * NS
EPW = N_EDGES // NW
EB = 80
NB = EPW // EB
RPW = N_NODES // NS
DWPW = 1000

_MESH = plsc.VectorSubcoreMesh(
    core_axis_name="c", subcore_axis_name="s", num_cores=NC, num_subcores=NS)


@functools.partial(
    pl.kernel,
    out_type=[jax.ShapeDtypeStruct((NC, N_NODES), jnp.float32),
              jax.ShapeDtypeStruct((NC, N_NODES), jnp.float32)],
    mesh=_MESH,
    scratch_types=[
        pltpu.VMEM((NB, EB), jnp.int32),
        pltpu.VMEM((NB, EB), jnp.int32),
        pltpu.VMEM((1024,), jnp.float32),
        pltpu.VMEM((EB,), jnp.float32),
        pltpu.VMEM_SHARED((N_NODES,), jnp.float32),
        pltpu.VMEM_SHARED((N_NODES,), jnp.float32),
    ],
)
def _degrees(src_hbm, dst_hbm, dgo_out, dgi_out,
             src_v, dst_v, zeros_v, ones_v, dgo_acc, dgi_acc):
    c = lax.axis_index("c")
    s = lax.axis_index("s")
    w = c * NS + s

    def fill_zeros(i, carry):
        zeros_v[pl.ds(i * 16, 16)] = jnp.zeros((16,), jnp.float32)
        return carry
    lax.fori_loop(0, 1024 // 16, fill_zeros, 0)

    def fill_ones(i, carry):
        ones_v[pl.ds(i * 16, 16)] = jnp.ones((16,), jnp.float32)
        return carry
    lax.fori_loop(0, EB // 16, fill_ones, 0)

    @pl.when(s < N_NODES // DWPW)
    def _():
        pltpu.sync_copy(zeros_v.at[pl.ds(0, DWPW)],
                        dgo_acc.at[pl.ds(s * DWPW, DWPW)])
        pltpu.sync_copy(zeros_v.at[pl.ds(0, DWPW)],
                        dgi_acc.at[pl.ds(s * DWPW, DWPW)])
    plsc.subcore_barrier()

    pltpu.sync_copy(src_hbm.at[w], src_v)
    pltpu.sync_copy(dst_hbm.at[w], dst_v)

    def body(j, carry):
        pltpu.sync_copy(ones_v, dgo_acc.at[src_v.at[j]], add=True)
        pltpu.sync_copy(ones_v, dgi_acc.at[dst_v.at[j]], add=True)
        return carry
    lax.fori_loop(0, NB, body, 0)
    plsc.subcore_barrier()

    @pl.when(s < N_NODES // DWPW)
    def _():
        pltpu.sync_copy(dgo_acc.at[pl.ds(s * DWPW, DWPW)],
                        dgo_out.at[c, pl.ds(s * DWPW, DWPW)])
        pltpu.sync_copy(dgi_acc.at[pl.ds(s * DWPW, DWPW)],
                        dgi_out.at[c, pl.ds(s * DWPW, DWPW)])


@functools.partial(
    pl.kernel,
    out_type=jax.ShapeDtypeStruct((NC, N_NODES, D_IN), jnp.float32),
    mesh=_MESH,
    scratch_types=[
        pltpu.VMEM((NB, EB), jnp.int32),
        pltpu.VMEM((NB, EB), jnp.int32),
        pltpu.VMEM((EB, D_IN), jnp.float32),
        pltpu.VMEM_SHARED((N_NODES, D_IN), jnp.float32),
    ],
)
def _spmm(table_hbm, src_hbm, dst_hbm, out_hbm, src_v, dst_v, rows_v, acc):
    c = lax.axis_index("c")
    s = lax.axis_index("s")
    w = c * NS + s

    def fill_zeros(i, carry):
        rows_v[i // 8, pl.ds((i % 8) * 16, 16)] = jnp.zeros((16,), jnp.float32)
        return carry
    lax.fori_loop(0, EB * (D_IN // 16), fill_zeros, 0)

    base = s * RPW
    for t in range(RPW // EB):
        pltpu.sync_copy(rows_v, acc.at[pl.ds(base + t * EB, EB)])
    rem = RPW % EB
    pltpu.sync_copy(rows_v.at[pl.ds(0, rem)],
                    acc.at[pl.ds(base + RPW - rem, rem)])
    plsc.subcore_barrier()

    pltpu.sync_copy(src_hbm.at[w], src_v)
    pltpu.sync_copy(dst_hbm.at[w], dst_v)

    def body(j, carry):
        pltpu.sync_copy(table_hbm.at[src_v.at[j]], rows_v)
        pltpu.sync_copy(rows_v, acc.at[dst_v.at[j]], add=True)
        return carry
    lax.fori_loop(0, NB, body, 0)
    plsc.subcore_barrier()

    pltpu.sync_copy(acc.at[pl.ds(base, RPW)],
                    out_hbm.at[c, pl.ds(base, RPW)])


def _prep_body(dgo_ref, dgi_ref, feat_ref, hs_ref, inn_ref, onn_ref):
    deg_o = dgo_ref[0] + dgo_ref[1]
    deg_i = dgi_ref[0] + dgi_ref[1]
    onn = lax.rsqrt(jnp.maximum(deg_o, 1.0))
    inn = lax.rsqrt(jnp.maximum(deg_i, 1.0))
    onn_ref[...] = onn
    inn_ref[...] = inn
    hs_ref[...] = feat_ref[...] * onn


_prep = pl.pallas_call(
    _prep_body,
    out_shape=[jax.ShapeDtypeStruct((N_NODES, D_IN), jnp.float32),
               jax.ShapeDtypeStruct((N_NODES, 1), jnp.float32),
               jax.ShapeDtypeStruct((N_NODES, 1), jnp.float32)],
)


def _mid_body(p_ref, inn_ref, onn_ref, w_ref, b_ref, o_ref):
    agg = (p_ref[0] + p_ref[1]) * inn_ref[...]
    y = jnp.dot(agg, w_ref[...], preferred_element_type=jnp.float32)
    o_ref[...] = jnp.maximum(y + b_ref[...][None, :], 0.0) * onn_ref[...]


_mid = pl.pallas_call(
    _mid_body,
    out_shape=jax.ShapeDtypeStruct((N_NODES, D_HID), jnp.float32),
)


def _final_body(p_ref, inn_ref, w_ref, b_ref, o_ref):
    agg = (p_ref[0] + p_ref[1]) * inn_ref[...]
    y = jnp.dot(agg, w_ref[...], preferred_element_type=jnp.float32)
    o_ref[...] = y + b_ref[...][None, :]


_final = pl.pallas_call(
    _final_body,
    out_shape=jax.ShapeDtypeStruct((N_NODES, D_OUT), jnp.float32),
)


def kernel(features, edge_index, W1, b1, W2, b2, W3, b3):
    src = edge_index[0].reshape(NW, NB, EB)
    dst = edge_index[1].reshape(NW, NB, EB)

    dgo_p, dgi_p = _degrees(src, dst)
    hs, inn, onn = _prep(dgo_p[:, :, None], dgi_p[:, :, None], features)

    p1 = _spmm(hs, src, dst)
    h1 = _mid(p1, inn, onn, W1, b1)
    p2 = _spmm(h1, src, dst)
    h2 = _mid(p2, inn, onn, W2, b2)
    p3 = _spmm(h2, src, dst)
    return _final(p3, inn, W3, b3)


# speedup vs baseline: 6.7606x; 6.7606x over previous
import functools

import jax
import jax.numpy as jnp
from jax import lax
from jax.experimental import pallas as pl
from jax.experimental.pallas import tpu as pltpu
from jax.experimental.pallas import tpu_sc as plsc

N_NODES = 10000
N_EDGES = 320000
D_IN = 128
D_HID = 128
D_OUT = 40

NC, NS = 2, 16
NW = NC * NS
EPW = N_EDGES // NW
EB = 80
NB = EPW // EB
RPW = 640
RPW_LAST = N_NODES - (NS - 1) * RPW
DWPW = 1000

_MESH = plsc.VectorSubcoreMesh(
    core_axis_name="c", subcore_axis_name="s", num_cores=NC, num_subcores=NS)


@functools.partial(
    pl.kernel,
    out_type=[jax.ShapeDtypeStruct((NC, 1, N_NODES), jnp.float32),
              jax.ShapeDtypeStruct((NC, 1, N_NODES), jnp.float32)],
    mesh=_MESH,
    scratch_types=[
        pltpu.VMEM((NB, EB), jnp.int32),
        pltpu.VMEM((NB, EB), jnp.int32),
        pltpu.VMEM((1024,), jnp.float32),
        pltpu.VMEM((EB,), jnp.float32),
        pltpu.VMEM_SHARED((N_NODES,), jnp.float32),
        pltpu.VMEM_SHARED((N_NODES,), jnp.float32),
    ],
)
def _degrees(src_hbm, dst_hbm, dgo_out, dgi_out,
             src_v, dst_v, zeros_v, ones_v, dgo_acc, dgi_acc):
    c = lax.axis_index("c")
    s = lax.axis_index("s")
    w = c * NS + s

    def fill_zeros(i, carry):
        zeros_v[pl.ds(i * 16, 16)] = jnp.zeros((16,), jnp.float32)
        return carry
    lax.fori_loop(0, 1024 // 16, fill_zeros, 0)

    def fill_ones(i, carry):
        ones_v[pl.ds(i * 16, 16)] = jnp.ones((16,), jnp.float32)
        return carry
    lax.fori_loop(0, EB // 16, fill_ones, 0)

    @pl.when(s < N_NODES // DWPW)
    def _():
        pltpu.sync_copy(zeros_v.at[pl.ds(0, DWPW)],
                        dgo_acc.at[pl.ds(s * DWPW, DWPW)])
        pltpu.sync_copy(zeros_v.at[pl.ds(0, DWPW)],
                        dgi_acc.at[pl.ds(s * DWPW, DWPW)])
    plsc.subcore_barrier()

    pltpu.sync_copy(src_hbm.at[w], src_v)
    pltpu.sync_copy(dst_hbm.at[w], dst_v)

    def body(j, carry):
        pltpu.sync_copy(ones_v, dgo_acc.at[src_v.at[j]], add=True)
        pltpu.sync_copy(ones_v, dgi_acc.at[dst_v.at[j]], add=True)
        return carry
    lax.fori_loop(0, NB, body, 0)
    plsc.subcore_barrier()

    @pl.when(s == 0)
    def _():
        pltpu.sync_copy(dgo_acc, dgo_out.at[c, 0])
        pltpu.sync_copy(dgi_acc, dgi_out.at[c, 0])


@functools.partial(
    pl.kernel,
    out_type=jax.ShapeDtypeStruct((NC, N_NODES, D_IN), jnp.float32),
    mesh=_MESH,
    scratch_types=[
        pltpu.VMEM((NB, EB), jnp.int32),
        pltpu.VMEM((NB, EB), jnp.int32),
        pltpu.VMEM((EB, D_IN), jnp.float32),
        pltpu.VMEM_SHARED((N_NODES, D_IN), jnp.float32),
    ],
)
def _spmm(table_hbm, src_hbm, dst_hbm, out_hbm, src_v, dst_v, rows_v, acc):
    c = lax.axis_index("c")
    s = lax.axis_index("s")
    w = c * NS + s

    def fill_zeros(i, carry):
        rows_v[i // 8, pl.ds((i % 8) * 16, 16)] = jnp.zeros((16,), jnp.float32)
        return carry
    lax.fori_loop(0, EB * (D_IN // 16), fill_zeros, 0)

    base = s * RPW

    @pl.when(s < NS - 1)
    def _():
        for t in range(RPW // EB):
            pltpu.sync_copy(rows_v, acc.at[pl.ds(base + t * EB, EB)])

    @pl.when(s == NS - 1)
    def _():
        for t in range(RPW_LAST // EB):
            pltpu.sync_copy(rows_v, acc.at[pl.ds(base + t * EB, EB)])
    plsc.subcore_barrier()

    pltpu.sync_copy(src_hbm.at[w], src_v)
    pltpu.sync_copy(dst_hbm.at[w], dst_v)

    def body(j, carry):
        pltpu.sync_copy(table_hbm.at[src_v.at[j]], rows_v)
        pltpu.sync_copy(rows_v, acc.at[dst_v.at[j]], add=True)
        return carry
    lax.fori_loop(0, NB, body, 0)
    plsc.subcore_barrier()

    @pl.when(s < NS - 1)
    def _():
        pltpu.sync_copy(acc.at[pl.ds(base, RPW)],
                        out_hbm.at[c, pl.ds(base, RPW)])

    @pl.when(s == NS - 1)
    def _():
        pltpu.sync_copy(acc.at[pl.ds(base, RPW_LAST)],
                        out_hbm.at[c, pl.ds(base, RPW_LAST)])


def _prep_body(dgo_ref, dgi_ref, feat_ref, hs_ref, inn_ref, onn_ref):
    deg_o = dgo_ref[0] + dgo_ref[1]
    deg_i = dgi_ref[0] + dgi_ref[1]
    onn = lax.rsqrt(jnp.maximum(deg_o, 1.0))
    inn = lax.rsqrt(jnp.maximum(deg_i, 1.0))
    onn_ref[...] = onn
    inn_ref[...] = inn
    hs_ref[...] = feat_ref[...] * onn


_prep = pl.pallas_call(
    _prep_body,
    out_shape=[jax.ShapeDtypeStruct((N_NODES, D_IN), jnp.float32),
               jax.ShapeDtypeStruct((N_NODES, 1), jnp.float32),
               jax.ShapeDtypeStruct((N_NODES, 1), jnp.float32)],
)


def _mid_body(p_ref, inn_ref, onn_ref, w_ref, b_ref, o_ref):
    agg = (p_ref[0] + p_ref[1]) * inn_ref[...]
    y = jnp.dot(agg, w_ref[...], preferred_element_type=jnp.float32)
    o_ref[...] = jnp.maximum(y + b_ref[...][None, :], 0.0) * onn_ref[...]


_mid = pl.pallas_call(
    _mid_body,
    out_shape=jax.ShapeDtypeStruct((N_NODES, D_HID), jnp.float32),
)


def _final_body(p_ref, inn_ref, w_ref, b_ref, o_ref):
    agg = (p_ref[0] + p_ref[1]) * inn_ref[...]
    y = jnp.dot(agg, w_ref[...], preferred_element_type=jnp.float32)
    o_ref[...] = y + b_ref[...][None, :]


_final = pl.pallas_call(
    _final_body,
    out_shape=jax.ShapeDtypeStruct((N_NODES, D_OUT), jnp.float32),
)


def kernel(features, edge_index, W1, b1, W2, b2, W3, b3):
    src = edge_index[0].reshape(NW, NB, EB)
    dst = edge_index[1].reshape(NW, NB, EB)

    dgo_p, dgi_p = _degrees(src, dst)
    hs, inn, onn = _prep(dgo_p[:, 0, :, None], dgi_p[:, 0, :, None], features)

    p1 = _spmm(hs, src, dst)
    h1 = _mid(p1, inn, onn, W1, b1)
    p2 = _spmm(h1, src, dst)
    h2 = _mid(p2, inn, onn, W2, b2)
    p3 = _spmm(h2, src, dst)
    return _final(p3, inn, W3, b3)


# speedup vs baseline: 10.4100x; 1.5398x over previous
import functools

import jax
import jax.numpy as jnp
from jax import lax
from jax.experimental import pallas as pl
from jax.experimental.pallas import tpu as pltpu
from jax.experimental.pallas import tpu_sc as plsc

N_NODES = 10000
N_EDGES = 320000
D_IN = 128
D_HID = 128
D_OUT = 40

NC, NS = 2, 16
NW = NC * NS
EPW = N_EDGES // NW
EB = 80
NB = EPW // EB
RPW = 640
RPW_LAST = N_NODES - (NS - 1) * RPW
DWPW = 1000

_MESH = plsc.VectorSubcoreMesh(
    core_axis_name="c", subcore_axis_name="s", num_cores=NC, num_subcores=NS)


@functools.partial(
    pl.kernel,
    out_type=[jax.ShapeDtypeStruct((NC, 1, N_NODES), jnp.float32),
              jax.ShapeDtypeStruct((NC, 1, N_NODES), jnp.float32)],
    mesh=_MESH,
    scratch_types=[
        pltpu.VMEM((NB, EB), jnp.int32),
        pltpu.VMEM((NB, EB), jnp.int32),
        pltpu.VMEM((1024,), jnp.float32),
        pltpu.VMEM((EB,), jnp.float32),
        pltpu.VMEM_SHARED((N_NODES,), jnp.float32),
        pltpu.VMEM_SHARED((N_NODES,), jnp.float32),
    ],
)
def _degrees(src_hbm, dst_hbm, dgo_out, dgi_out,
             src_v, dst_v, zeros_v, ones_v, dgo_acc, dgi_acc):
    c = lax.axis_index("c")
    s = lax.axis_index("s")
    w = c * NS + s

    def fill_zeros(i, carry):
        zeros_v[pl.ds(i * 16, 16)] = jnp.zeros((16,), jnp.float32)
        return carry
    lax.fori_loop(0, 1024 // 16, fill_zeros, 0)

    def fill_ones(i, carry):
        ones_v[pl.ds(i * 16, 16)] = jnp.ones((16,), jnp.float32)
        return carry
    lax.fori_loop(0, EB // 16, fill_ones, 0)

    @pl.when(s < N_NODES // DWPW)
    def _():
        pltpu.sync_copy(zeros_v.at[pl.ds(0, DWPW)],
                        dgo_acc.at[pl.ds(s * DWPW, DWPW)])
        pltpu.sync_copy(zeros_v.at[pl.ds(0, DWPW)],
                        dgi_acc.at[pl.ds(s * DWPW, DWPW)])
    plsc.subcore_barrier()

    pltpu.sync_copy(src_hbm.at[w], src_v)
    pltpu.sync_copy(dst_hbm.at[w], dst_v)

    def body(j, carry):
        pltpu.sync_copy(ones_v, dgo_acc.at[src_v.at[j]], add=True)
        pltpu.sync_copy(ones_v, dgi_acc.at[dst_v.at[j]], add=True)
        return carry
    lax.fori_loop(0, NB, body, 0)
    plsc.subcore_barrier()

    @pl.when(s == 0)
    def _():
        pltpu.sync_copy(dgo_acc, dgo_out.at[c, 0])
        pltpu.sync_copy(dgi_acc, dgi_out.at[c, 0])


@functools.partial(
    pl.kernel,
    out_type=jax.ShapeDtypeStruct((NC, N_NODES, D_IN), jnp.float32),
    mesh=_MESH,
    scratch_types=[
        pltpu.VMEM((EPW,), jnp.int32),
        pltpu.VMEM((NB, EB), jnp.int32),
        pltpu.VMEM((EB, D_IN), jnp.float32),
        pltpu.VMEM((EB, D_IN), jnp.float32),
        pltpu.VMEM_SHARED((N_NODES, D_IN), jnp.float32),
        pltpu.SemaphoreType.DMA,
        pltpu.SemaphoreType.DMA,
    ],
)
def _spmm(table_hbm, src_hbm, dst_hbm, out_hbm,
          src_v, dst_v, rows_a, rows_b, acc, sem_a, sem_b):
    c = lax.axis_index("c")
    s = lax.axis_index("s")
    w = c * NS + s

    def fill_zeros(i, carry):
        rows_a[i // 8, pl.ds((i % 8) * 16, 16)] = jnp.zeros((16,), jnp.float32)
        return carry
    lax.fori_loop(0, EB * (D_IN // 16), fill_zeros, 0)

    base = s * RPW

    @pl.when(s < NS - 1)
    def _():
        for t in range(RPW // EB):
            pltpu.sync_copy(rows_a, acc.at[pl.ds(base + t * EB, EB)])

    @pl.when(s == NS - 1)
    def _():
        for t in range(RPW_LAST // EB):
            pltpu.sync_copy(rows_a, acc.at[pl.ds(base + t * EB, EB)])
    plsc.subcore_barrier()

    pltpu.sync_copy(src_hbm.at[pl.ds(w * EPW, EPW)], src_v)
    pltpu.sync_copy(dst_hbm.at[w], dst_v)

    def _gather(j, buf, sem):
        return pltpu.async_copy(
            table_hbm.at[src_v.at[pl.ds(j * EB, EB)]], buf, sem)

    def _gather_wait(j, buf, sem):
        pltpu.make_async_copy(
            table_hbm.at[src_v.at[pl.ds(j * EB, EB)]], buf, sem).wait()

    _gather(0, rows_a, sem_a)

    def body(jj, carry):
        j = 2 * jj
        cp_b = _gather(j + 1, rows_b, sem_b)
        _gather_wait(j, rows_a, sem_a)
        pltpu.sync_copy(rows_a, acc.at[dst_v.at[j]], add=True)
        _gather(j + 2, rows_a, sem_a)
        cp_b.wait()
        pltpu.sync_copy(rows_b, acc.at[dst_v.at[j + 1]], add=True)
        return carry
    lax.fori_loop(0, (NB - 1) // 2, body, 0)
    _gather_wait(NB - 1, rows_a, sem_a)
    pltpu.sync_copy(rows_a, acc.at[dst_v.at[NB - 1]], add=True)
    plsc.subcore_barrier()

    @pl.when(s < NS - 1)
    def _():
        pltpu.sync_copy(acc.at[pl.ds(base, RPW)],
                        out_hbm.at[c, pl.ds(base, RPW)])

    @pl.when(s == NS - 1)
    def _():
        pltpu.sync_copy(acc.at[pl.ds(base, RPW_LAST)],
                        out_hbm.at[c, pl.ds(base, RPW_LAST)])


def _prep_body(dgo_ref, dgi_ref, feat_ref, hs_ref, inn_ref, onn_ref):
    deg_o = dgo_ref[0] + dgo_ref[1]
    deg_i = dgi_ref[0] + dgi_ref[1]
    onn = lax.rsqrt(jnp.maximum(deg_o, 1.0))
    inn = lax.rsqrt(jnp.maximum(deg_i, 1.0))
    onn_ref[...] = onn
    inn_ref[...] = inn
    hs_ref[...] = feat_ref[...] * onn


_prep = pl.pallas_call(
    _prep_body,
    out_shape=[jax.ShapeDtypeStruct((N_NODES, D_IN), jnp.float32),
               jax.ShapeDtypeStruct((N_NODES, 1), jnp.float32),
               jax.ShapeDtypeStruct((N_NODES, 1), jnp.float32)],
)


def _mid_body(p_ref, inn_ref, onn_ref, w_ref, b_ref, o_ref):
    agg = (p_ref[0] + p_ref[1]) * inn_ref[...]
    y = jnp.dot(agg, w_ref[...], preferred_element_type=jnp.float32)
    o_ref[...] = jnp.maximum(y + b_ref[...][None, :], 0.0) * onn_ref[...]


_mid = pl.pallas_call(
    _mid_body,
    out_shape=jax.ShapeDtypeStruct((N_NODES, D_HID), jnp.float32),
)


def _final_body(p_ref, inn_ref, w_ref, b_ref, o_ref):
    agg = (p_ref[0] + p_ref[1]) * inn_ref[...]
    y = jnp.dot(agg, w_ref[...], preferred_element_type=jnp.float32)
    o_ref[...] = y + b_ref[...][None, :]


_final = pl.pallas_call(
    _final_body,
    out_shape=jax.ShapeDtypeStruct((N_NODES, D_OUT), jnp.float32),
)


def kernel(features, edge_index, W1, b1, W2, b2, W3, b3):
    src_flat = edge_index[0]
    src = edge_index[0].reshape(NW, NB, EB)
    dst = edge_index[1].reshape(NW, NB, EB)

    dgo_p, dgi_p = _degrees(src, dst)
    hs, inn, onn = _prep(dgo_p[:, 0, :, None], dgi_p[:, 0, :, None], features)

    p1 = _spmm(hs, src_flat, dst)
    h1 = _mid(p1, inn, onn, W1, b1)
    p2 = _spmm(h1, src_flat, dst)
    h2 = _mid(p2, inn, onn, W2, b2)
    p3 = _spmm(h2, src_flat, dst)
    return _final(p3, inn, W3, b3)
